# Initial kernel scaffold; baseline (speedup 1.0000x reference)
import jax
import jax.numpy as jnp
from jax.experimental import pallas as pl


def kernel(code_levels, table_0, table_1, table_2, table_3):
    raise NotImplementedError("write your pallas kernel here")


# speedup vs baseline: 3.4446x; 3.4446x over previous
import functools

import jax
import jax.numpy as jnp
from jax import lax
from jax.experimental import pallas as pl
from jax.experimental.pallas import tpu as pltpu
from jax.experimental.pallas import tpu_sc as plsc

N = 100000
NUM_WORKERS = 32
PER_W = 3128
NPAD = NUM_WORKERS * PER_W
C = 128
NFULL = PER_W // C
TAIL = PER_W - NFULL * C
DIMS = (16, 32, 32, 48)
COLS = (0, 16, 48, 80)
DOUT = 128

_mesh = plsc.VectorSubcoreMesh(core_axis_name="c", subcore_axis_name="s")


@functools.partial(
    pl.kernel,
    out_type=jax.ShapeDtypeStruct((NPAD, DOUT), jnp.float32),
    mesh=_mesh,
    scratch_types=(
        [pltpu.VMEM((C,), jnp.int32) for _ in range(4)]
        + [pltpu.VMEM((C, d), jnp.float32) for d in DIMS]
        + [pltpu.VMEM((TAIL,), jnp.int32) for _ in range(4)]
        + [pltpu.VMEM((TAIL, d), jnp.float32) for d in DIMS]
        + [pltpu.SemaphoreType.DMA]
    ),
    compiler_params=pltpu.CompilerParams(use_tc_tiling_on_sc=False),
)
def _sc_lookup(
    idx0, idx1, idx2, idx3,
    t0, t1, t2, t3,
    out,
    iv0, iv1, iv2, iv3,
    rv0, rv1, rv2, rv3,
    tiv0, tiv1, tiv2, tiv3,
    trv0, trv1, trv2, trv3,
    sem,
):
    idxs = (idx0, idx1, idx2, idx3)
    tables = (t0, t1, t2, t3)
    ivs = (iv0, iv1, iv2, iv3)
    rvs = (rv0, rv1, rv2, rv3)
    tivs = (tiv0, tiv1, tiv2, tiv3)
    trvs = (trv0, trv1, trv2, trv3)

    wid = lax.axis_index("s") * 2 + lax.axis_index("c")
    base = wid * PER_W

    def do_chunk(off, n, iv, rv):
        for l in range(4):
            pltpu.sync_copy(idxs[l].at[pl.ds(off, n)], iv[l])
        copies = [
            pltpu.async_copy(tables[l].at[iv[l]], rv[l], sem) for l in range(4)
        ]
        for cp in copies:
            cp.wait()
        for l in range(4):
            pltpu.sync_copy(
                rv[l], out.at[pl.ds(off, n), pl.ds(COLS[l], DIMS[l])]
            )

    @pl.loop(0, NFULL)
    def _(j):
        do_chunk(base + j * C, C, ivs, rvs)

    do_chunk(base + NFULL * C, TAIL, tivs, trvs)


def kernel(code_levels, table_0, table_1, table_2, table_3):
    cl = jnp.pad(code_levels, ((0, NPAD - N), (0, 0)))
    out = _sc_lookup(
        cl[:, 0], cl[:, 1], cl[:, 2], cl[:, 3],
        table_0, table_1, table_2, table_3,
    )
    return out[:N]


# speedup vs baseline: 3.8403x; 1.1149x over previous
import functools

import jax
import jax.numpy as jnp
from jax import lax
from jax.experimental import pallas as pl
from jax.experimental.pallas import tpu as pltpu
from jax.experimental.pallas import tpu_sc as plsc

N = 100000
NUM_WORKERS = 32
PER_W = 3128
NPAD = NUM_WORKERS * PER_W
C = 128
NFULL = PER_W // C
TAIL = PER_W - NFULL * C
K = 4
NGROUPS = NFULL // K
DIMS = (16, 32, 32, 48)
COLS = (0, 16, 48, 80)
DOUT = 128

_mesh = plsc.VectorSubcoreMesh(core_axis_name="c", subcore_axis_name="s")

_scratch = []
for _k in range(K):
    _scratch.append(pltpu.VMEM((4, C), jnp.int32))
    _scratch.extend(pltpu.VMEM((C, d), jnp.float32) for d in DIMS)
_scratch.append(pltpu.VMEM((4, TAIL), jnp.int32))
_scratch.extend(pltpu.VMEM((TAIL, d), jnp.float32) for d in DIMS)
_scratch.extend(pltpu.SemaphoreType.DMA for _ in range(K + 1))


@functools.partial(
    pl.kernel,
    out_type=jax.ShapeDtypeStruct((NPAD, DOUT), jnp.float32),
    mesh=_mesh,
    scratch_types=_scratch,
    compiler_params=pltpu.CompilerParams(use_tc_tiling_on_sc=False),
)
def _sc_lookup(idx0, idx1, idx2, idx3, t0, t1, t2, t3, out, *s):
    idxs = (idx0, idx1, idx2, idx3)
    tables = (t0, t1, t2, t3)
    slots = []
    p = 0
    for _ in range(K):
        slots.append((s[p], list(s[p + 1:p + 5])))
        p += 5
    tiv, trvs = s[p], list(s[p + 1:p + 5])
    p += 5
    sems = s[p:p + K]
    tsem = s[p + K]

    wid = lax.axis_index("s") * 2 + lax.axis_index("c")
    base = wid * PER_W

    def stage_and_fire(off, n, iv, rvs, sem):
        for l in range(4):
            pltpu.sync_copy(idxs[l].at[pl.ds(off, n)], iv.at[l])
        for l in range(4):
            pltpu.async_copy(tables[l].at[iv.at[l]], rvs[l], sem)

    def drain(rvs, sem, tabs):
        for l in range(4):
            pltpu.make_async_copy(tabs[l], rvs[l], sem).wait()

    def write_out(off, n, rvs):
        for l in range(4):
            pltpu.sync_copy(rvs[l], out.at[pl.ds(off, n), pl.ds(COLS[l], DIMS[l])])

    for k in range(K):
        iv, rvs = slots[k]
        stage_and_fire(base + k * C, C, iv, rvs, sems[k])
    stage_and_fire(base + NFULL * C, TAIL, tiv, trvs, tsem)

    @pl.loop(0, NGROUPS)
    def _(g):
        for k in range(K):
            jj = g * K + k
            iv, rvs = slots[k]
            drain(rvs, sems[k], [tables[l].at[iv.at[l]] for l in range(4)])
            write_out(base + jj * C, C, rvs)

            @pl.when(jj + K < NFULL)
            def _():
                stage_and_fire(base + (jj + K) * C, C, iv, rvs, sems[k])

    drain(trvs, tsem, [tables[l].at[tiv.at[l]] for l in range(4)])
    write_out(base + NFULL * C, TAIL, trvs)


def kernel(code_levels, table_0, table_1, table_2, table_3):
    cl = jnp.pad(code_levels, ((0, NPAD - N), (0, 0)))
    out = _sc_lookup(
        cl[:, 0], cl[:, 1], cl[:, 2], cl[:, 3],
        table_0, table_1, table_2, table_3,
    )
    return out[:N]


# speedup vs baseline: 3.9281x; 1.0229x over previous
import functools

import jax
import jax.numpy as jnp
from jax import lax
from jax.experimental import pallas as pl
from jax.experimental.pallas import tpu as pltpu
from jax.experimental.pallas import tpu_sc as plsc

N = 100000
NUM_WORKERS = 32
PER_W = 3128
NPAD = NUM_WORKERS * PER_W
C = 128
NFULL = PER_W // C
TAIL = PER_W - NFULL * C
K = 6
NGROUPS = NFULL // K
DIMS = (16, 32, 32, 48)
COLS = (0, 16, 48, 80)
DOUT = 128

_mesh = plsc.VectorSubcoreMesh(core_axis_name="c", subcore_axis_name="s")

_scratch = [pltpu.VMEM((4, PER_W), jnp.int32)]
for _k in range(K):
    _scratch.extend(pltpu.VMEM((C, d), jnp.float32) for d in DIMS)
_scratch.extend(pltpu.VMEM((TAIL, d), jnp.float32) for d in DIMS)
_scratch.extend(pltpu.SemaphoreType.DMA for _ in range(K + 1))


@functools.partial(
    pl.kernel,
    out_type=jax.ShapeDtypeStruct((NPAD, DOUT), jnp.float32),
    mesh=_mesh,
    scratch_types=_scratch,
    compiler_params=pltpu.CompilerParams(use_tc_tiling_on_sc=False),
)
def _sc_lookup(idx0, idx1, idx2, idx3, t0, t1, t2, t3, out, *s):
    idxs = (idx0, idx1, idx2, idx3)
    tables = (t0, t1, t2, t3)
    iv = s[0]
    slots = [list(s[1 + 4 * k:5 + 4 * k]) for k in range(K)]
    trvs = list(s[1 + 4 * K:5 + 4 * K])
    sems = s[5 + 4 * K:5 + 5 * K]
    tsem = s[5 + 5 * K]

    wid = lax.axis_index("s") * 2 + lax.axis_index("c")
    base = wid * PER_W

    for l in range(4):
        pltpu.sync_copy(idxs[l].at[pl.ds(base, PER_W)], iv.at[l])

    def fire(off, n, rvs, sem):
        for l in range(4):
            pltpu.async_copy(tables[l].at[iv.at[l, pl.ds(off, n)]], rvs[l], sem)

    def drain(off, n, rvs, sem):
        for l in range(4):
            pltpu.make_async_copy(
                tables[l].at[iv.at[l, pl.ds(off, n)]], rvs[l], sem
            ).wait()

    def write_out(off, n, rvs):
        for l in range(4):
            pltpu.sync_copy(
                rvs[l], out.at[pl.ds(base + off, n), pl.ds(COLS[l], DIMS[l])]
            )

    for k in range(K):
        fire(k * C, C, slots[k], sems[k])
    fire(NFULL * C, TAIL, trvs, tsem)

    @pl.loop(0, NGROUPS)
    def _(g):
        for k in range(K):
            jj = g * K + k
            off = jj * C
            drain(off, C, slots[k], sems[k])
            write_out(off, C, slots[k])

            @pl.when(jj + K < NFULL)
            def _():
                fire(off + K * C, C, slots[k], sems[k])

    drain(NFULL * C, TAIL, trvs, tsem)
    write_out(NFULL * C, TAIL, trvs)


def kernel(code_levels, table_0, table_1, table_2, table_3):
    cl = jnp.pad(code_levels, ((0, NPAD - N), (0, 0)))
    out = _sc_lookup(
        cl[:, 0], cl[:, 1], cl[:, 2], cl[:, 3],
        table_0, table_1, table_2, table_3,
    )
    return out[:N]
